# baseline (device time: 99367 ns/iter reference)
import math

import jax
import jax.numpy as jnp
from jax import lax
from jax.experimental import pallas as pl
from jax.experimental.pallas import tpu as pltpu

N_DEV = 8


def kernel(q, k, v):
    s_per, d = q.shape
    half = s_per // 2
    scale = 1.0 / math.sqrt(d)

    def body(q_ref, k_ref, v_ref, o_ref,
             own16, abuf, bbuf, zbuf, cbuf, dbuf, diag, fdiag,
             send_sems, recv_sems):
        my = lax.axis_index("i")
        base = (my // 4) * 4
        rel = my - base
        p_right = base + lax.rem(rel + 1, 4)
        p_left = base + lax.rem(rel + 3, 4)
        z_mirror = lax.rem(my + 4, N_DEV)

        barrier_sem = pltpu.get_barrier_semaphore()
        for nbr in (p_right, p_left, z_mirror):
            pl.semaphore_signal(
                barrier_sem, inc=1,
                device_id=(nbr,), device_id_type=pl.DeviceIdType.MESH,
            )
        pl.semaphore_wait(barrier_sem, 3)

        own16[0, :, :] = k_ref[:, :].astype(jnp.bfloat16)
        own16[1, :, :] = v_ref[:, :].astype(jnp.bfloat16)

        qb = (q_ref[:, :] * scale).astype(jnp.bfloat16)

        ck_sz = s_per // 4

        def partial(buf, m, l, acc):
            for ck in range(4):
                kb = buf[0, ck * ck_sz:(ck + 1) * ck_sz, :]
                vb = buf[1, ck * ck_sz:(ck + 1) * ck_sz, :]
                s = lax.dot_general(
                    qb, kb, (((1,), (1,)), ((), ())),
                    preferred_element_type=jnp.float32,
                )
                m_new = jnp.maximum(m, jnp.max(s, axis=1, keepdims=True))
                p = jnp.exp(s - m_new)
                corr = jnp.exp(m - m_new)
                l = l * corr + jnp.sum(p, axis=1, keepdims=True)
                acc = acc * corr + lax.dot_general(
                    p.astype(jnp.bfloat16), vb, (((1,), (0,)), ((), ())),
                    preferred_element_type=jnp.float32,
                )
                m = m_new
            return m, l, acc

        def send(src, dst, sem_i, target):
            rdma = pltpu.make_async_remote_copy(
                src_ref=src, dst_ref=dst,
                send_sem=send_sems.at[sem_i],
                recv_sem=recv_sems.at[sem_i],
                device_id=(target,),
                device_id_type=pl.DeviceIdType.MESH,
            )
            rdma.start()
            return rdma

        m = jnp.full((s_per, 1), -1e30, jnp.float32)
        l = jnp.zeros((s_per, 1), jnp.float32)
        acc = jnp.zeros((s_per, d), jnp.float32)

        sa = send(own16, abuf, 0, p_right)
        sb = send(own16, bbuf, 1, p_left)
        sz = send(own16, zbuf, 2, z_mirror)
        m, l, acc = partial(own16, m, l, acc)
        sa.wait_recv()
        sb.wait_recv()
        sz.wait_recv()

        sc = send(zbuf, cbuf, 3, p_right)
        sd = send(zbuf, dbuf, 4, p_left)
        se = send(abuf.at[:, 0:half, :], diag.at[:, 0:half, :], 5, p_right)
        sf = send(bbuf.at[:, half:, :], diag.at[:, half:, :], 6, p_left)
        m, l, acc = partial(abuf, m, l, acc)
        m, l, acc = partial(bbuf, m, l, acc)
        m, l, acc = partial(zbuf, m, l, acc)
        sc.wait_recv()
        sd.wait_recv()
        se.wait_recv()
        sf.wait_recv()

        sg = send(cbuf.at[:, 0:half, :], fdiag.at[:, 0:half, :], 7, p_right)
        sh = send(dbuf.at[:, half:, :], fdiag.at[:, half:, :], 8, p_left)
        m, l, acc = partial(cbuf, m, l, acc)
        m, l, acc = partial(dbuf, m, l, acc)
        m, l, acc = partial(diag, m, l, acc)
        sg.wait_recv()
        sh.wait_recv()
        m, l, acc = partial(fdiag, m, l, acc)

        for rd in (sa, sb, sz, sc, sd, se, sf, sg, sh):
            rd.wait_send()

        o_ref[:, :] = acc / l

    kv_shape = (2, s_per, d)
    return pl.pallas_call(
        body,
        out_shape=jax.ShapeDtypeStruct((s_per, d), jnp.float32),
        in_specs=[pl.BlockSpec(memory_space=pltpu.VMEM)] * 3,
        out_specs=pl.BlockSpec(memory_space=pltpu.VMEM),
        scratch_shapes=[
            pltpu.VMEM(kv_shape, jnp.bfloat16),
            pltpu.VMEM(kv_shape, jnp.bfloat16),
            pltpu.VMEM(kv_shape, jnp.bfloat16),
            pltpu.VMEM(kv_shape, jnp.bfloat16),
            pltpu.VMEM(kv_shape, jnp.bfloat16),
            pltpu.VMEM(kv_shape, jnp.bfloat16),
            pltpu.VMEM(kv_shape, jnp.bfloat16),
            pltpu.VMEM(kv_shape, jnp.bfloat16),
            pltpu.SemaphoreType.DMA((9,)),
            pltpu.SemaphoreType.DMA((9,)),
        ],
        compiler_params=pltpu.CompilerParams(
            collective_id=0,
            vmem_limit_bytes=100 * 1024 * 1024,
        ),
    )(q, k, v)


# device time: 98727 ns/iter; 1.0065x vs baseline; 1.0065x over previous
import math

import jax
import jax.numpy as jnp
from jax import lax
from jax.experimental import pallas as pl
from jax.experimental.pallas import tpu as pltpu

N_DEV = 8


def kernel(q, k, v):
    s_per, d = q.shape
    half = s_per // 2
    scale = 1.0 / math.sqrt(d)

    def body(q_ref, k_ref, v_ref, o_ref,
             own16, abuf, bbuf, zbuf, cbuf, dbuf, diag, fdiag,
             send_sems, recv_sems):
        my = lax.axis_index("i")
        base = (my // 4) * 4
        rel = my - base
        p_right = base + lax.rem(rel + 1, 4)
        p_left = base + lax.rem(rel + 3, 4)
        z_mirror = lax.rem(my + 4, N_DEV)

        barrier_sem = pltpu.get_barrier_semaphore()
        for nbr in (p_right, p_left, z_mirror):
            pl.semaphore_signal(
                barrier_sem, inc=1,
                device_id=(nbr,), device_id_type=pl.DeviceIdType.MESH,
            )
        pl.semaphore_wait(barrier_sem, 3)

        own16[0, :, :] = k_ref[:, :].astype(jnp.bfloat16)
        own16[1, :, :] = v_ref[:, :].astype(jnp.bfloat16)

        qb = (q_ref[:, :] * scale).astype(jnp.bfloat16)

        ck_sz = s_per // 4

        def partial(buf, m, l, acc):
            for ck in range(4):
                kb = buf[0, ck * ck_sz:(ck + 1) * ck_sz, :]
                vb = buf[1, ck * ck_sz:(ck + 1) * ck_sz, :]
                s = lax.dot_general(
                    qb, kb, (((1,), (1,)), ((), ())),
                    preferred_element_type=jnp.float32,
                )
                m_new = jnp.maximum(m, jnp.max(s, axis=1, keepdims=True))
                p = jnp.exp(s - m_new)
                corr = jnp.exp(m - m_new)
                l = l * corr + jnp.sum(p, axis=1, keepdims=True)
                acc = acc * corr + lax.dot_general(
                    p, vb, (((1,), (0,)), ((), ())),
                    preferred_element_type=jnp.float32,
                )
                m = m_new
            return m, l, acc

        def send(src, dst, sem_i, target):
            rdma = pltpu.make_async_remote_copy(
                src_ref=src, dst_ref=dst,
                send_sem=send_sems.at[sem_i],
                recv_sem=recv_sems.at[sem_i],
                device_id=(target,),
                device_id_type=pl.DeviceIdType.MESH,
            )
            rdma.start()
            return rdma

        m = jnp.full((s_per, 1), -1e30, jnp.float32)
        l = jnp.zeros((s_per, 1), jnp.float32)
        acc = jnp.zeros((s_per, d), jnp.float32)

        sa = send(own16, abuf, 0, p_right)
        sb = send(own16, bbuf, 1, p_left)
        sz = send(own16, zbuf, 2, z_mirror)
        m, l, acc = partial(own16, m, l, acc)
        sa.wait_recv()
        sb.wait_recv()
        sz.wait_recv()

        sc = send(zbuf, cbuf, 3, p_right)
        sd = send(zbuf, dbuf, 4, p_left)
        se = send(abuf.at[:, 0:half, :], diag.at[:, 0:half, :], 5, p_right)
        sf = send(bbuf.at[:, half:, :], diag.at[:, half:, :], 6, p_left)
        m, l, acc = partial(abuf, m, l, acc)
        m, l, acc = partial(bbuf, m, l, acc)
        m, l, acc = partial(zbuf, m, l, acc)
        sc.wait_recv()
        sd.wait_recv()
        se.wait_recv()
        sf.wait_recv()

        sg = send(cbuf.at[:, 0:half, :], fdiag.at[:, 0:half, :], 7, p_right)
        sh = send(dbuf.at[:, half:, :], fdiag.at[:, half:, :], 8, p_left)
        m, l, acc = partial(cbuf, m, l, acc)
        m, l, acc = partial(dbuf, m, l, acc)
        m, l, acc = partial(diag, m, l, acc)
        sg.wait_recv()
        sh.wait_recv()
        m, l, acc = partial(fdiag, m, l, acc)

        for rd in (sa, sb, sz, sc, sd, se, sf, sg, sh):
            rd.wait_send()

        o_ref[:, :] = acc / l

    kv_shape = (2, s_per, d)
    return pl.pallas_call(
        body,
        out_shape=jax.ShapeDtypeStruct((s_per, d), jnp.float32),
        in_specs=[pl.BlockSpec(memory_space=pltpu.VMEM)] * 3,
        out_specs=pl.BlockSpec(memory_space=pltpu.VMEM),
        scratch_shapes=[
            pltpu.VMEM(kv_shape, jnp.bfloat16),
            pltpu.VMEM(kv_shape, jnp.bfloat16),
            pltpu.VMEM(kv_shape, jnp.bfloat16),
            pltpu.VMEM(kv_shape, jnp.bfloat16),
            pltpu.VMEM(kv_shape, jnp.bfloat16),
            pltpu.VMEM(kv_shape, jnp.bfloat16),
            pltpu.VMEM(kv_shape, jnp.bfloat16),
            pltpu.VMEM(kv_shape, jnp.bfloat16),
            pltpu.SemaphoreType.DMA((9,)),
            pltpu.SemaphoreType.DMA((9,)),
        ],
        compiler_params=pltpu.CompilerParams(
            collective_id=0,
            vmem_limit_bytes=100 * 1024 * 1024,
        ),
    )(q, k, v)
